# Initial kernel scaffold; baseline (speedup 1.0000x reference)
import jax
import jax.numpy as jnp
from jax.experimental import pallas as pl


def kernel(shapes, colors, clusters, mask, shape_table, color_table, cluster_table, W1, b1, W2, b2):
    raise NotImplementedError("write your pallas kernel here")


# speedup vs baseline: 6.7506x; 6.7506x over previous
import functools

import jax
import jax.numpy as jnp
from jax import lax
from jax.experimental import pallas as pl
from jax.experimental.pallas import tpu as pltpu
from jax.experimental.pallas import tpu_sc as plsc

B = 4096
L = 200
D = 32
NC = 2
NS = 16
NW = NC * NS
RPW = B // NW
NB = 8
NCHUNK = RPW // NB
WIN = ((0, 104), (104, 96))


def _pool_body(shapes_hbm, colors_hbm, clusters_hbm, mask_hbm,
               st_hbm, ct_hbm, kt_hbm, out_hbm,
               idx_s, idx_c, idx_k, mask_v,
               rows_s, rows_c, rows_k, out_v,
               sem_s, sem_c, sem_k):
    wid = lax.axis_index("subcore") * NC + lax.axis_index("core")
    base = wid * RPW

    @pl.loop(0, NCHUNK)
    def _(chunk):
        b0 = base + chunk * NB
        pltpu.sync_copy(shapes_hbm.at[pl.ds(b0, NB)], idx_s)
        pltpu.sync_copy(colors_hbm.at[pl.ds(b0, NB)], idx_c)
        pltpu.sync_copy(clusters_hbm.at[pl.ds(b0, NB)], idx_k)
        pltpu.sync_copy(mask_hbm.at[pl.ds(b0, NB)], mask_v)

        @pl.loop(0, NB)
        def _(bi):
            accs = (jnp.zeros((16,), jnp.float32),) * 6
            for off, nw in WIN:
                sl = pl.ds(off, nw)
                d1 = pltpu.async_copy(st_hbm.at[idx_s.at[bi, sl]],
                                      rows_s.at[pl.ds(0, nw)], sem_s)
                d2 = pltpu.async_copy(ct_hbm.at[idx_c.at[bi, sl]],
                                      rows_c.at[pl.ds(0, nw)], sem_c)
                d3 = pltpu.async_copy(kt_hbm.at[idx_k.at[bi, sl]],
                                      rows_k.at[pl.ds(0, nw)], sem_k)
                d1.wait()
                d2.wait()
                d3.wait()

                def step(l0, carry, nl, _off=off, _bi=bi):
                    mchunk = mask_v[_bi, pl.ds(_off + l0, 16)]
                    a0, a1, a2, a3, a4, a5 = carry
                    for i in range(nl):
                        m = jnp.broadcast_to(mchunk[i], (16,))
                        l = l0 + i
                        a0 = a0 + m * rows_s[l, 0:16]
                        a1 = a1 + m * rows_s[l, 16:32]
                        a2 = a2 + m * rows_c[l, 0:16]
                        a3 = a3 + m * rows_c[l, 16:32]
                        a4 = a4 + m * rows_k[l, 0:16]
                        a5 = a5 + m * rows_k[l, 16:32]
                    return (a0, a1, a2, a3, a4, a5)

                ngr, tail = nw // 16, nw % 16
                accs = lax.fori_loop(
                    0, ngr, lambda g, c: step(g * 16, c, 16), accs)
                if tail:
                    accs = step(ngr * 16, accs, tail)
            for j in range(6):
                out_v[bi, 16 * j:16 * (j + 1)] = accs[j]

        pltpu.sync_copy(out_v, out_hbm.at[pl.ds(b0, NB)])


def _pooled_sums(shapes, colors, clusters, mask,
                 shape_table, color_table, cluster_table):
    mesh = plsc.VectorSubcoreMesh(core_axis_name="core",
                                  subcore_axis_name="subcore")
    f = pl.kernel(
        _pool_body,
        out_type=jax.ShapeDtypeStruct((B, 3 * D), jnp.float32),
        mesh=mesh,
        compiler_params=pltpu.CompilerParams(use_tc_tiling_on_sc=False),
        scratch_types=[
            pltpu.VMEM((NB, L), jnp.int32),
            pltpu.VMEM((NB, L), jnp.int32),
            pltpu.VMEM((NB, L), jnp.int32),
            pltpu.VMEM((NB, L), jnp.float32),
            pltpu.VMEM((WIN[0][1], D), jnp.float32),
            pltpu.VMEM((WIN[0][1], D), jnp.float32),
            pltpu.VMEM((WIN[0][1], D), jnp.float32),
            pltpu.VMEM((NB, 3 * D), jnp.float32),
            pltpu.SemaphoreType.DMA,
            pltpu.SemaphoreType.DMA,
            pltpu.SemaphoreType.DMA,
        ],
    )
    return f(shapes, colors, clusters, mask,
             shape_table, color_table, cluster_table)


def _mlp_body(sums_ref, mask_ref, w1_ref, b1_ref, w2_ref, b2_ref, out_ref):
    denom = jnp.sum(mask_ref[...], axis=1, keepdims=True)
    pooled = sums_ref[...] / denom
    h = jnp.dot(pooled, w1_ref[...], preferred_element_type=jnp.float32,
                precision=lax.Precision.HIGHEST) + b1_ref[...]
    h = jnp.maximum(h, 0.0)
    out_ref[...] = jnp.dot(h, w2_ref[...], preferred_element_type=jnp.float32,
                           precision=lax.Precision.HIGHEST) + b2_ref[...]


def _mlp(sums, mask, W1, b1, W2, b2):
    bm = 512
    grid = (B // bm,)
    return pl.pallas_call(
        _mlp_body,
        grid=grid,
        in_specs=[
            pl.BlockSpec((bm, 3 * D), lambda i: (i, 0)),
            pl.BlockSpec((bm, L), lambda i: (i, 0)),
            pl.BlockSpec(W1.shape, lambda i: (0, 0)),
            pl.BlockSpec((1, b1.shape[0]), lambda i: (0, 0)),
            pl.BlockSpec(W2.shape, lambda i: (0, 0)),
            pl.BlockSpec((1, b2.shape[0]), lambda i: (0, 0)),
        ],
        out_specs=pl.BlockSpec((bm, b2.shape[0]), lambda i: (i, 0)),
        out_shape=jax.ShapeDtypeStruct((B, b2.shape[0]), jnp.float32),
    )(sums, mask, W1, b1.reshape(1, -1), W2, b2.reshape(1, -1))


def kernel(shapes, colors, clusters, mask, shape_table, color_table,
           cluster_table, W1, b1, W2, b2):
    sums = _pooled_sums(shapes, colors, clusters, mask,
                        shape_table, color_table, cluster_table)
    return _mlp(sums, mask, W1, b1, W2, b2)


# speedup vs baseline: 8.4471x; 1.2513x over previous
import functools

import jax
import jax.numpy as jnp
from jax import lax
from jax.experimental import pallas as pl
from jax.experimental.pallas import tpu as pltpu
from jax.experimental.pallas import tpu_sc as plsc

B = 4096
L = 200
D = 32
NC = 2
NS = 16
NW = NC * NS
RPW = B // NW
NB = 32
NCHUNK = RPW // NB
NPAIR = NB // 2
WIN = ((0, 104), (104, 96))


def _pool_body(shapes_hbm, colors_hbm, clusters_hbm, mask_hbm,
               st_hbm, ct_hbm, kt_hbm, out_hbm,
               idx_s, idx_c, idx_k, mask_v,
               rows_s0, rows_c0, rows_k0, rows_s1, rows_c1, rows_k1,
               out_v,
               sem_s0, sem_c0, sem_k0, sem_s1, sem_c1, sem_k1):
    wid = lax.axis_index("subcore") * NC + lax.axis_index("core")
    base = wid * RPW

    slots = (
        ((rows_s0, rows_c0, rows_k0), (sem_s0, sem_c0, sem_k0)),
        ((rows_s1, rows_c1, rows_k1), (sem_s1, sem_c1, sem_k1)),
    )
    tables = (st_hbm, ct_hbm, kt_hbm)
    idxs = (idx_s, idx_c, idx_k)

    def issue(bi, slot):
        bufs, sems = slots[slot]
        for t in range(3):
            for off, nw in WIN:
                pltpu.async_copy(
                    tables[t].at[idxs[t].at[bi, pl.ds(off, nw)]],
                    bufs[t].at[pl.ds(off, nw)], sems[t])

    def drain(bi, slot):
        bufs, sems = slots[slot]
        for t in range(3):
            for off, nw in WIN:
                pltpu.make_async_copy(
                    tables[t].at[idxs[t].at[bi, pl.ds(off, nw)]],
                    bufs[t].at[pl.ds(off, nw)], sems[t]).wait()

    def accumulate(bi, slot):
        (rs, rc, rk), _ = slots[slot]
        accs = (jnp.zeros((16,), jnp.float32),) * 6

        def step(l0, carry, nl):
            mchunk = mask_v[bi, pl.ds(l0, 16)]
            a0, a1, a2, a3, a4, a5 = carry
            for i in range(nl):
                m = jnp.broadcast_to(mchunk[i], (16,))
                l = l0 + i
                a0 = a0 + m * rs[l, 0:16]
                a1 = a1 + m * rs[l, 16:32]
                a2 = a2 + m * rc[l, 0:16]
                a3 = a3 + m * rc[l, 16:32]
                a4 = a4 + m * rk[l, 0:16]
                a5 = a5 + m * rk[l, 16:32]
            return (a0, a1, a2, a3, a4, a5)

        for off, nw in WIN:
            ngr, tail = nw // 16, nw % 16
            accs = lax.fori_loop(
                0, ngr,
                functools.partial(
                    lambda g, c, _o: step(_o + g * 16, c, 16), _o=off),
                accs)
            if tail:
                accs = step(off + ngr * 16, accs, tail)
        for j in range(6):
            out_v[bi, 16 * j:16 * (j + 1)] = accs[j]

    @pl.loop(0, NCHUNK)
    def _(chunk):
        b0 = base + chunk * NB
        pltpu.sync_copy(shapes_hbm.at[pl.ds(b0, NB)], idx_s)
        pltpu.sync_copy(colors_hbm.at[pl.ds(b0, NB)], idx_c)
        pltpu.sync_copy(clusters_hbm.at[pl.ds(b0, NB)], idx_k)
        pltpu.sync_copy(mask_hbm.at[pl.ds(b0, NB)], mask_v)

        issue(0, 0)

        @pl.loop(0, NPAIR - 1)
        def _(p):
            r0 = 2 * p
            issue(r0 + 1, 1)
            drain(r0, 0)
            accumulate(r0, 0)
            issue(r0 + 2, 0)
            drain(r0 + 1, 1)
            accumulate(r0 + 1, 1)

        issue(NB - 1, 1)
        drain(NB - 2, 0)
        accumulate(NB - 2, 0)
        drain(NB - 1, 1)
        accumulate(NB - 1, 1)

        pltpu.sync_copy(out_v, out_hbm.at[pl.ds(b0, NB)])


def _pooled_sums(shapes, colors, clusters, mask,
                 shape_table, color_table, cluster_table):
    mesh = plsc.VectorSubcoreMesh(core_axis_name="core",
                                  subcore_axis_name="subcore")
    f = pl.kernel(
        _pool_body,
        out_type=jax.ShapeDtypeStruct((B, 3 * D), jnp.float32),
        mesh=mesh,
        compiler_params=pltpu.CompilerParams(use_tc_tiling_on_sc=False),
        scratch_types=[
            pltpu.VMEM((NB, L), jnp.int32),
            pltpu.VMEM((NB, L), jnp.int32),
            pltpu.VMEM((NB, L), jnp.int32),
            pltpu.VMEM((NB, L), jnp.float32),
            pltpu.VMEM((L, D), jnp.float32),
            pltpu.VMEM((L, D), jnp.float32),
            pltpu.VMEM((L, D), jnp.float32),
            pltpu.VMEM((L, D), jnp.float32),
            pltpu.VMEM((L, D), jnp.float32),
            pltpu.VMEM((L, D), jnp.float32),
            pltpu.VMEM((NB, 3 * D), jnp.float32),
            pltpu.SemaphoreType.DMA,
            pltpu.SemaphoreType.DMA,
            pltpu.SemaphoreType.DMA,
            pltpu.SemaphoreType.DMA,
            pltpu.SemaphoreType.DMA,
            pltpu.SemaphoreType.DMA,
        ],
    )
    return f(shapes, colors, clusters, mask,
             shape_table, color_table, cluster_table)


def _mlp_body(sums_ref, mask_ref, w1_ref, b1_ref, w2_ref, b2_ref, out_ref):
    denom = jnp.sum(mask_ref[...], axis=1, keepdims=True)
    pooled = sums_ref[...] / denom
    h = jnp.dot(pooled, w1_ref[...], preferred_element_type=jnp.float32,
                precision=lax.Precision.HIGHEST) + b1_ref[...]
    h = jnp.maximum(h, 0.0)
    out_ref[...] = jnp.dot(h, w2_ref[...], preferred_element_type=jnp.float32,
                           precision=lax.Precision.HIGHEST) + b2_ref[...]


def _mlp(sums, mask, W1, b1, W2, b2):
    bm = 512
    grid = (B // bm,)
    return pl.pallas_call(
        _mlp_body,
        grid=grid,
        in_specs=[
            pl.BlockSpec((bm, 3 * D), lambda i: (i, 0)),
            pl.BlockSpec((bm, L), lambda i: (i, 0)),
            pl.BlockSpec(W1.shape, lambda i: (0, 0)),
            pl.BlockSpec((1, b1.shape[0]), lambda i: (0, 0)),
            pl.BlockSpec(W2.shape, lambda i: (0, 0)),
            pl.BlockSpec((1, b2.shape[0]), lambda i: (0, 0)),
        ],
        out_specs=pl.BlockSpec((bm, b2.shape[0]), lambda i: (i, 0)),
        out_shape=jax.ShapeDtypeStruct((B, b2.shape[0]), jnp.float32),
    )(sums, mask, W1, b1.reshape(1, -1), W2, b2.reshape(1, -1))


def kernel(shapes, colors, clusters, mask, shape_table, color_table,
           cluster_table, W1, b1, W2, b2):
    sums = _pooled_sums(shapes, colors, clusters, mask,
                        shape_table, color_table, cluster_table)
    return _mlp(sums, mask, W1, b1, W2, b2)
